# baseline (speedup 1.0000x reference)
import functools

import jax
import jax.numpy as jnp
from jax import lax
from jax.experimental import pallas as pl
from jax.experimental.pallas import tpu as pltpu
from jax.experimental.pallas import tpu_sc as plsc

NC = 2
NS = 16
NW = NC * NS
LANES = 16

D = 768
BATCH = 4
SEQ = 8192
SPW = SEQ // NW
CP = 64
NCH = SPW // CP
VPR = D // LANES


def _emb_kernel(x_hbm, tab_hbm, pos_hbm, out_hbm, idx_v, pos_v, tok_v, sem):
    wid = lax.axis_index("s") * NC + lax.axis_index("c")
    s0 = wid * SPW

    for c in range(NCH):
        for b in range(BATCH):
            pltpu.sync_copy(
                x_hbm.at[pl.ds(b * SEQ + s0 + c * CP, CP)],
                idx_v.at[c * BATCH + b],
            )

    def add_row(r, carry):
        for v in range(VPR):
            sl = pl.ds(v * LANES, LANES)
            tok_v[r, sl] = tok_v[r, sl] + pos_v[r, sl]
        return carry

    for c in range(NCH):
        pltpu.sync_copy(pos_hbm.at[pl.ds(s0 + c * CP, CP), :], pos_v)
        for b in range(BATCH):
            pltpu.async_copy(tab_hbm.at[idx_v.at[c * BATCH + b]], tok_v, sem).wait()
            lax.fori_loop(0, CP, add_row, 0)
            pltpu.sync_copy(
                tok_v, out_hbm.at[pl.ds(b * SEQ + s0 + c * CP, CP), :]
            )


@jax.jit
def _emb(x_flat, tab, pos):
    mesh = plsc.VectorSubcoreMesh(
        core_axis_name="c", subcore_axis_name="s", num_cores=NC, num_subcores=NS
    )
    run = functools.partial(
        pl.kernel,
        out_type=jax.ShapeDtypeStruct((BATCH * SEQ, D), jnp.float32),
        mesh=mesh,
        scratch_types=[
            pltpu.VMEM((NCH * BATCH, CP), jnp.int32),
            pltpu.VMEM((CP, D), jnp.float32),
            pltpu.VMEM((CP, D), jnp.float32),
            pltpu.SemaphoreType.DMA,
        ],
    )(_emb_kernel)
    return run(x_flat, tab, pos)


def kernel(x, tok_table, pos_emb):
    x_flat = x.reshape(-1).astype(jnp.int32)
    pos = pos_emb[: x.shape[1], :]
    out = _emb(x_flat, tok_table, pos)
    return out.reshape(x.shape[0], x.shape[1], D)
